# baseline (speedup 1.0000x reference)
import functools

import numpy as np

import jax
import jax.numpy as jnp
from jax import lax
from jax.experimental import pallas as pl
from jax.experimental.pallas import tpu as pltpu
from jax.experimental.pallas import tpu_sc as plsc

N = 10000
NP = 10112
H = 256
HW = 128
CW = 16
E = 160000
EL = 20000

NC = 2
NS = 16
B = 128
SG = 16
ST = 5
G = ST * SG
EP = NS * G * B
ELG = 5
ELP = NC * NS * ELG * B
RPT = NP // NS
RCHUNKS = ((0, 128), (128, 128), (256, 128), (384, 128), (512, 120))

_mesh = plsc.VectorSubcoreMesh(core_axis_name="c", subcore_axis_name="s")

_IPERM = np.empty((H,), np.int32)
for _k in range(H // 32):
    for _i in range(16):
        _IPERM[32 * _k + 2 * _i] = 32 * _k + _i
        _IPERM[32 * _k + 2 * _i + 1] = 32 * _k + 16 + _i


def _bf_halves(x):
    xb = x[:, _IPERM].astype(jnp.bfloat16)
    xi = jax.lax.bitcast_convert_type(xb.reshape(N, H // 2, 2), jnp.int32)
    return xi[:, :HW // 2], xi[:, HW // 2:]


def _agg_body(x0_hbm, x1_hbm, gidx_hbm, sidx_hbm, zeros_hbm, sums_out,
              gidx_v, sidx_v, rbf_0, rbf_1, rows_f, sums_sp,
              sem_00, sem_01, sem_10, sem_11):
    c = lax.axis_index("c")
    s = lax.axis_index("s")
    r0 = s * RPT

    pltpu.sync_copy(zeros_hbm.at[pl.ds(0, B)], rows_f)
    for off, sz in RCHUNKS:
        pltpu.sync_copy(rows_f.at[pl.ds(0, sz)],
                        sums_sp.at[pl.ds(r0 + off, sz)])
    plsc.subcore_barrier()

    def run(x2d, ci):
        HB = B // 2
        bufs = (rbf_0, rbf_1)
        sems = ((sem_00, sem_01), (sem_10, sem_11))

        def fire(g, bi):
            buf, (s1, s2) = bufs[bi], sems[bi]
            pltpu.async_copy(x2d.at[gidx_v.at[g, pl.ds(0, HB)]],
                             buf.at[pl.ds(0, HB)], s1)
            pltpu.async_copy(x2d.at[gidx_v.at[g, pl.ds(HB, HB)]],
                             buf.at[pl.ds(HB, HB)], s2)

        def drain(g, bi):
            buf, (s1, s2) = bufs[bi], sems[bi]
            pltpu.make_async_copy(x2d.at[gidx_v.at[g, pl.ds(0, HB)]],
                                  buf.at[pl.ds(0, HB)], s1).wait()
            pltpu.make_async_copy(x2d.at[gidx_v.at[g, pl.ds(HB, HB)]],
                                  buf.at[pl.ds(HB, HB)], s2).wait()

        def convert(bi):
            buf = bufs[bi]

            def crow(q, carry3):
                for j in range(2):
                    r = 2 * q + j
                    for k in range(HW // 32):
                        x = buf[r, pl.ds(16 * k, 16)]
                        lo = jax.lax.bitcast_convert_type(x << 16, jnp.float32)
                        hi = jax.lax.bitcast_convert_type(
                            x & jnp.int32(-65536), jnp.float32)
                        rows_f[r, pl.ds(32 * k, 16)] = lo
                        rows_f[r, pl.ds(32 * k + 16, 16)] = hi
                return carry3

            lax.fori_loop(0, B // 2, crow, 0)

        def stage(t, carry):
            pltpu.sync_copy(gidx_hbm.at[s, pl.ds(t * SG, SG)], gidx_v)
            pltpu.sync_copy(sidx_hbm.at[s, pl.ds(t * SG, SG)], sidx_v)
            fire(0, 0)

            def pair(p, carry2):
                g0 = 2 * p
                fire(g0 + 1, 1)
                drain(g0, 0)
                convert(0)
                pltpu.sync_copy(rows_f, sums_sp.at[sidx_v.at[g0]], add=True)

                @pl.when(p < SG // 2 - 1)
                def _():
                    fire(g0 + 2, 0)

                drain(g0 + 1, 1)
                convert(1)
                pltpu.sync_copy(rows_f, sums_sp.at[sidx_v.at[g0 + 1]],
                                add=True)
                return carry2

            lax.fori_loop(0, SG // 2, pair, 0)
            return carry

        lax.fori_loop(0, ST, stage, 0)
        plsc.subcore_barrier()
        for off, sz in RCHUNKS:
            pltpu.sync_copy(sums_sp.at[pl.ds(r0 + off, sz)],
                            rows_f.at[pl.ds(0, sz)])
            pltpu.sync_copy(rows_f.at[pl.ds(0, sz)],
                            sums_out.at[ci, pl.ds(r0 + off, sz)])

    @pl.when(c == 0)
    def _():
        run(x0_hbm, 0)

    @pl.when(c == 1)
    def _():
        run(x1_hbm, 1)


_agg_sc = pl.kernel(
    _agg_body,
    out_type=jax.ShapeDtypeStruct((NC, NP, HW), jnp.float32),
    mesh=_mesh,
    compiler_params=pltpu.CompilerParams(use_tc_tiling_on_sc=False),
    scratch_types=[
        pltpu.VMEM((SG, B), jnp.int32),
        pltpu.VMEM((SG, B), jnp.int32),
        pltpu.VMEM((B, HW // 2), jnp.int32),
        pltpu.VMEM((B, HW // 2), jnp.int32),
        pltpu.VMEM((B, HW), jnp.float32),
        pltpu.VMEM_SHARED((NP, HW), jnp.float32),
        pltpu.SemaphoreType.DMA,
        pltpu.SemaphoreType.DMA,
        pltpu.SemaphoreType.DMA,
        pltpu.SemaphoreType.DMA,
    ],
)


def _cnt_body(didx_hbm, uidx_hbm, zeros_hbm, ones_hbm, cnt_out,
              idx_v, ones_v, cnt_sp):
    c = lax.axis_index("c")
    s = lax.axis_index("s")
    r0 = s * RPT

    pltpu.sync_copy(zeros_hbm.at[pl.ds(0, B)], ones_v)
    for off, sz in RCHUNKS:
        pltpu.sync_copy(ones_v.at[pl.ds(0, sz)],
                        cnt_sp.at[pl.ds(r0 + off, sz)])
    pltpu.sync_copy(ones_hbm, ones_v)
    plsc.subcore_barrier()

    def run(eidx_hbm, ci):
        def stage(t, carry):
            pltpu.sync_copy(eidx_hbm.at[s, pl.ds(t * SG, SG)], idx_v)

            def chunk(g, carry2):
                pltpu.sync_copy(ones_v, cnt_sp.at[idx_v.at[g]], add=True)
                return carry2

            lax.fori_loop(0, SG, chunk, 0)
            return carry

        lax.fori_loop(0, ST, stage, 0)
        plsc.subcore_barrier()
        for off, sz in RCHUNKS:
            pltpu.sync_copy(cnt_sp.at[pl.ds(r0 + off, sz)],
                            ones_v.at[pl.ds(0, sz)])
            pltpu.sync_copy(ones_v.at[pl.ds(0, sz)],
                            cnt_out.at[ci, pl.ds(r0 + off, sz)])

    @pl.when(c == 0)
    def _():
        run(didx_hbm, 0)

    @pl.when(c == 1)
    def _():
        run(uidx_hbm, 1)


_cnt_sc = pl.kernel(
    _cnt_body,
    out_type=jax.ShapeDtypeStruct((NC, NP, HW), jnp.float32),
    mesh=_mesh,
    scratch_types=[
        pltpu.VMEM((SG, B), jnp.int32),
        pltpu.VMEM((B, HW), jnp.float32),
        pltpu.VMEM_SHARED((NP, HW), jnp.float32),
    ],
)


def _dot_body(hu0, hu1, hm0, hm1, aidx_hbm, bidx_hbm, out_hbm,
              aidx_v, bidx_v, a0, a1, b0, b1, p_v, sem):
    c = lax.axis_index("c")
    s = lax.axis_index("s")
    wid = s * NC + c
    pltpu.sync_copy(aidx_hbm.at[wid], aidx_v)
    pltpu.sync_copy(bidx_hbm.at[wid], bidx_v)
    mask = jnp.int32(-65536)

    def chunk(g, carry):
        cp0 = pltpu.async_copy(hu0.at[aidx_v.at[g]], a0, sem)
        cp1 = pltpu.async_copy(hu1.at[aidx_v.at[g]], a1, sem)
        cp2 = pltpu.async_copy(hm0.at[bidx_v.at[g]], b0, sem)
        cp3 = pltpu.async_copy(hm1.at[bidx_v.at[g]], b1, sem)
        cp0.wait(); cp1.wait(); cp2.wait(); cp3.wait()

        def row(r, carry2):
            for k in range(HW // 32):
                sl = pl.ds(16 * k, 16)
                xa0, xb0 = a0[r, sl], b0[r, sl]
                xa1, xb1 = a1[r, sl], b1[r, sl]
                lo = (jax.lax.bitcast_convert_type(xa0 << 16, jnp.float32)
                      * jax.lax.bitcast_convert_type(xb0 << 16, jnp.float32)
                      + jax.lax.bitcast_convert_type(xa1 << 16, jnp.float32)
                      * jax.lax.bitcast_convert_type(xb1 << 16, jnp.float32))
                hi = (jax.lax.bitcast_convert_type(xa0 & mask, jnp.float32)
                      * jax.lax.bitcast_convert_type(xb0 & mask, jnp.float32)
                      + jax.lax.bitcast_convert_type(xa1 & mask, jnp.float32)
                      * jax.lax.bitcast_convert_type(xb1 & mask, jnp.float32))
                p_v[r, pl.ds(32 * k, 16)] = lo
                p_v[r, pl.ds(32 * k + 16, 16)] = hi
            return carry2

        lax.fori_loop(0, B, row, 0)
        pltpu.sync_copy(p_v, out_hbm.at[pl.ds(wid * (ELG * B) + g * B, B)])
        return carry

    lax.fori_loop(0, ELG, chunk, 0)


_dot_sc = pl.kernel(
    _dot_body,
    out_type=jax.ShapeDtypeStruct((ELP, HW), jnp.float32),
    mesh=_mesh,
    compiler_params=pltpu.CompilerParams(use_tc_tiling_on_sc=False),
    scratch_types=[
        pltpu.VMEM((ELG, B), jnp.int32),
        pltpu.VMEM((ELG, B), jnp.int32),
        pltpu.VMEM((B, HW // 2), jnp.int32),
        pltpu.VMEM((B, HW // 2), jnp.int32),
        pltpu.VMEM((B, HW // 2), jnp.int32),
        pltpu.VMEM((B, HW // 2), jnp.int32),
        pltpu.VMEM((B, HW), jnp.float32),
        pltpu.SemaphoreType.DMA,
    ],
)


def _rowsum_body(p_ref, o_ref):
    o_ref[...] = jnp.sum(p_ref[...], axis=1, keepdims=True)


def _rowsum_tc(p):
    RB = 2048
    return pl.pallas_call(
        _rowsum_body,
        grid=(ELP // RB,),
        in_specs=[pl.BlockSpec((RB, HW), lambda i: (i, 0))],
        out_specs=pl.BlockSpec((RB, 1), lambda i: (i, 0)),
        out_shape=jax.ShapeDtypeStruct((ELP, 1), jnp.float32),
    )(p)


def _sage_body(relu, x_ref, s0_ref, s1_ref, cnt_ref, ws_ref, wn_ref, b_ref,
               o_ref):
    inv = 1.0 / jnp.maximum(cnt_ref[0, :, 0:1], 1.0)
    agg = jnp.concatenate([s0_ref[0], s1_ref[0]], axis=1) * inv
    y = (jnp.dot(x_ref[...], ws_ref[...], preferred_element_type=jnp.float32)
         + jnp.dot(agg, wn_ref[...], preferred_element_type=jnp.float32)
         + b_ref[...])
    o_ref[...] = jnp.maximum(y, 0.0) if relu else y


def _sage_tc(x, S, cnt, hsel, ws, wn, b, relu):
    RB = 1000
    return pl.pallas_call(
        functools.partial(_sage_body, relu),
        grid=(N // RB,),
        in_specs=[
            pl.BlockSpec((RB, H), lambda i: (i, 0)),
            pl.BlockSpec((1, RB, HW), lambda i: (0, i, 0)),
            pl.BlockSpec((1, RB, HW), lambda i: (1, i, 0)),
            pl.BlockSpec((1, RB, HW), lambda i, hsel=hsel: (hsel, i, 0)),
            pl.BlockSpec((H, H), lambda i: (0, 0)),
            pl.BlockSpec((H, H), lambda i: (0, 0)),
            pl.BlockSpec((1, H), lambda i: (0, 0)),
        ],
        out_specs=pl.BlockSpec((RB, H), lambda i: (i, 0)),
        out_shape=jax.ShapeDtypeStruct((N, H), jnp.float32),
    )(x, S, S, cnt, ws, wn, b.reshape(1, H))


def kernel(user_emb, movie_emb, W_self_um1, W_nei_um1, W_self_mu1, W_nei_mu1,
           W_self_um2, W_nei_um2, W_self_mu2, W_nei_mu2, b_m1, b_u1, b_m2,
           b_u2, edge_index, edge_label_index):
    src = edge_index[0].astype(jnp.int32)
    dst = edge_index[1].astype(jnp.int32)
    pad_g = jnp.zeros((EP - E,), jnp.int32)
    pad_s = jnp.full((EP - E,), N, jnp.int32)
    src_g = jnp.concatenate([src, pad_g]).reshape(NS, G, B)
    src_s = jnp.concatenate([src, pad_s]).reshape(NS, G, B)
    dst_g = jnp.concatenate([dst, pad_g]).reshape(NS, G, B)
    dst_s = jnp.concatenate([dst, pad_s]).reshape(NS, G, B)
    zeros = jnp.zeros((NP, HW), jnp.float32)
    ones = jnp.ones((B, HW), jnp.float32)

    u0, u1 = _bf_halves(user_emb)
    m0, m1 = _bf_halves(movie_emb)

    cnt = _cnt_sc(dst_s, src_s, zeros, ones)

    S_m = _agg_sc(u0, u1, src_g, dst_s, zeros)
    S_u = _agg_sc(m0, m1, dst_g, src_s, zeros)
    h_m = _sage_tc(movie_emb, S_m, cnt, 0, W_self_um1, W_nei_um1, b_m1, True)
    h_u = _sage_tc(user_emb, S_u, cnt, 1, W_self_mu1, W_nei_mu1, b_u1, True)

    hu0, hu1 = _bf_halves(h_u)
    hm0, hm1 = _bf_halves(h_m)
    S_m2 = _agg_sc(hu0, hu1, src_g, dst_s, zeros)
    S_u2 = _agg_sc(hm0, hm1, dst_g, src_s, zeros)
    h_m2 = _sage_tc(h_m, S_m2, cnt, 0, W_self_um2, W_nei_um2, b_m2, False)
    h_u2 = _sage_tc(h_u, S_u2, cnt, 1, W_self_mu2, W_nei_mu2, b_u2, False)
    hu2_0, hu2_1 = _bf_halves(h_u2)
    hm2_0, hm2_1 = _bf_halves(h_m2)

    eli_pad = jnp.zeros((ELP - EL,), jnp.int32)
    aidx = jnp.concatenate([edge_label_index[0].astype(jnp.int32), eli_pad])
    bidx = jnp.concatenate([edge_label_index[1].astype(jnp.int32), eli_pad])
    prod = _dot_sc(hu2_0, hu2_1, hm2_0, hm2_1,
                   aidx.reshape(NC * NS, ELG, B), bidx.reshape(NC * NS, ELG, B))
    pred = _rowsum_tc(prod)
    return pred[:EL, 0]


# speedup vs baseline: 1.0663x; 1.0663x over previous
import functools

import numpy as np

import jax
import jax.numpy as jnp
from jax import lax
from jax.experimental import pallas as pl
from jax.experimental.pallas import tpu as pltpu
from jax.experimental.pallas import tpu_sc as plsc

N = 10000
NP = 10112
H = 256
HW = 128
CW = 16
E = 160000
EL = 20000

NC = 2
NS = 16
B = 128
SG = 16
ST = 5
G = ST * SG
EP = NS * G * B
ELG = 5
ELP = NC * NS * ELG * B
RPT = NP // NS
RCHUNKS = ((0, 128), (128, 128), (256, 128), (384, 128), (512, 120))

_mesh = plsc.VectorSubcoreMesh(core_axis_name="c", subcore_axis_name="s")

_IPERM = np.empty((H,), np.int32)
for _k in range(H // 32):
    for _i in range(16):
        _IPERM[32 * _k + 2 * _i] = 32 * _k + _i
        _IPERM[32 * _k + 2 * _i + 1] = 32 * _k + 16 + _i


def _bf_halves(x):
    xb = x[:, _IPERM].astype(jnp.bfloat16)
    xi = jax.lax.bitcast_convert_type(xb.reshape(N, H // 2, 2), jnp.int32)
    return xi[:, :HW // 2], xi[:, HW // 2:]


def _agg_body(x0_hbm, x1_hbm, gidx_hbm, sidx_hbm, zeros_hbm, sums_out,
              gidx_v, sidx_v, rbf_0, rbf_1, rows_f, sums_sp,
              sem_00, sem_01, sem_10, sem_11):
    c = lax.axis_index("c")
    s = lax.axis_index("s")
    r0 = s * RPT

    pltpu.sync_copy(zeros_hbm.at[pl.ds(0, B)], rows_f)
    for off, sz in RCHUNKS:
        pltpu.sync_copy(rows_f.at[pl.ds(0, sz)],
                        sums_sp.at[pl.ds(r0 + off, sz)])
    plsc.subcore_barrier()

    def run(x2d, ci):
        HB = B // 2
        bufs = (rbf_0, rbf_1)
        sems = ((sem_00, sem_01), (sem_10, sem_11))

        def fire(g, bi):
            buf, (s1, s2) = bufs[bi], sems[bi]
            pltpu.async_copy(x2d.at[gidx_v.at[g, pl.ds(0, HB)]],
                             buf.at[pl.ds(0, HB)], s1)
            pltpu.async_copy(x2d.at[gidx_v.at[g, pl.ds(HB, HB)]],
                             buf.at[pl.ds(HB, HB)], s2)

        def drain(g, bi):
            buf, (s1, s2) = bufs[bi], sems[bi]
            pltpu.make_async_copy(x2d.at[gidx_v.at[g, pl.ds(0, HB)]],
                                  buf.at[pl.ds(0, HB)], s1).wait()
            pltpu.make_async_copy(x2d.at[gidx_v.at[g, pl.ds(HB, HB)]],
                                  buf.at[pl.ds(HB, HB)], s2).wait()

        def convert(bi):
            buf = bufs[bi]

            def crow(r, carry3):
                for k in range(HW // 32):
                    x = buf[r, pl.ds(16 * k, 16)]
                    lo = jax.lax.bitcast_convert_type(x << 16, jnp.float32)
                    hi = jax.lax.bitcast_convert_type(
                        x & jnp.int32(-65536), jnp.float32)
                    rows_f[r, pl.ds(32 * k, 16)] = lo
                    rows_f[r, pl.ds(32 * k + 16, 16)] = hi
                return carry3

            lax.fori_loop(0, B, crow, 0)

        def stage(t, carry):
            pltpu.sync_copy(gidx_hbm.at[s, pl.ds(t * SG, SG)], gidx_v)
            pltpu.sync_copy(sidx_hbm.at[s, pl.ds(t * SG, SG)], sidx_v)
            fire(0, 0)

            def pair(p, carry2):
                g0 = 2 * p
                fire(g0 + 1, 1)
                drain(g0, 0)
                convert(0)
                pltpu.sync_copy(rows_f, sums_sp.at[sidx_v.at[g0]], add=True)

                @pl.when(p < SG // 2 - 1)
                def _():
                    fire(g0 + 2, 0)

                drain(g0 + 1, 1)
                convert(1)
                pltpu.sync_copy(rows_f, sums_sp.at[sidx_v.at[g0 + 1]],
                                add=True)
                return carry2

            lax.fori_loop(0, SG // 2, pair, 0)
            return carry

        lax.fori_loop(0, ST, stage, 0)
        plsc.subcore_barrier()
        for off, sz in RCHUNKS:
            pltpu.sync_copy(sums_sp.at[pl.ds(r0 + off, sz)],
                            rows_f.at[pl.ds(0, sz)])
            pltpu.sync_copy(rows_f.at[pl.ds(0, sz)],
                            sums_out.at[ci, pl.ds(r0 + off, sz)])

    @pl.when(c == 0)
    def _():
        run(x0_hbm, 0)

    @pl.when(c == 1)
    def _():
        run(x1_hbm, 1)


_agg_sc = pl.kernel(
    _agg_body,
    out_type=jax.ShapeDtypeStruct((NC, NP, HW), jnp.float32),
    mesh=_mesh,
    compiler_params=pltpu.CompilerParams(use_tc_tiling_on_sc=False),
    scratch_types=[
        pltpu.VMEM((SG, B), jnp.int32),
        pltpu.VMEM((SG, B), jnp.int32),
        pltpu.VMEM((B, HW // 2), jnp.int32),
        pltpu.VMEM((B, HW // 2), jnp.int32),
        pltpu.VMEM((B, HW), jnp.float32),
        pltpu.VMEM_SHARED((NP, HW), jnp.float32),
        pltpu.SemaphoreType.DMA,
        pltpu.SemaphoreType.DMA,
        pltpu.SemaphoreType.DMA,
        pltpu.SemaphoreType.DMA,
    ],
)


def _cnt_body(didx_hbm, uidx_hbm, zeros_hbm, ones_hbm, cnt_out,
              idx_v, ones_v, cnt_sp):
    c = lax.axis_index("c")
    s = lax.axis_index("s")
    r0 = s * RPT

    pltpu.sync_copy(zeros_hbm.at[pl.ds(0, B)], ones_v)
    for off, sz in RCHUNKS:
        pltpu.sync_copy(ones_v.at[pl.ds(0, sz)],
                        cnt_sp.at[pl.ds(r0 + off, sz)])
    pltpu.sync_copy(ones_hbm, ones_v)
    plsc.subcore_barrier()

    def run(eidx_hbm, ci):
        def stage(t, carry):
            pltpu.sync_copy(eidx_hbm.at[s, pl.ds(t * SG, SG)], idx_v)

            def chunk(g, carry2):
                pltpu.sync_copy(ones_v, cnt_sp.at[idx_v.at[g]], add=True)
                return carry2

            lax.fori_loop(0, SG, chunk, 0)
            return carry

        lax.fori_loop(0, ST, stage, 0)
        plsc.subcore_barrier()
        for off, sz in RCHUNKS:
            pltpu.sync_copy(cnt_sp.at[pl.ds(r0 + off, sz)],
                            ones_v.at[pl.ds(0, sz)])
            pltpu.sync_copy(ones_v.at[pl.ds(0, sz)],
                            cnt_out.at[ci, pl.ds(r0 + off, sz)])

    @pl.when(c == 0)
    def _():
        run(didx_hbm, 0)

    @pl.when(c == 1)
    def _():
        run(uidx_hbm, 1)


_cnt_sc = pl.kernel(
    _cnt_body,
    out_type=jax.ShapeDtypeStruct((NC, NP, HW), jnp.float32),
    mesh=_mesh,
    scratch_types=[
        pltpu.VMEM((SG, B), jnp.int32),
        pltpu.VMEM((B, HW), jnp.float32),
        pltpu.VMEM_SHARED((NP, HW), jnp.float32),
    ],
)


def _dot_body(hu0, hu1, hm0, hm1, aidx_hbm, bidx_hbm, out_hbm,
              aidx_v, bidx_v, a0, a1, b0, b1, p_v, sem):
    c = lax.axis_index("c")
    s = lax.axis_index("s")
    wid = s * NC + c
    pltpu.sync_copy(aidx_hbm.at[wid], aidx_v)
    pltpu.sync_copy(bidx_hbm.at[wid], bidx_v)

    def chunk(g, carry):
        cp0 = pltpu.async_copy(hu0.at[aidx_v.at[g]], a0, sem)
        cp1 = pltpu.async_copy(hu1.at[aidx_v.at[g]], a1, sem)
        cp2 = pltpu.async_copy(hm0.at[bidx_v.at[g]], b0, sem)
        cp3 = pltpu.async_copy(hm1.at[bidx_v.at[g]], b1, sem)
        cp0.wait(); cp1.wait(); cp2.wait(); cp3.wait()

        def row(r, carry2):
            for k in range(HW // 16):
                sl = pl.ds(k * 16, 16)
                p_v[r, sl] = a0[r, sl] * b0[r, sl] + a1[r, sl] * b1[r, sl]
            return carry2

        lax.fori_loop(0, B, row, 0)
        pltpu.sync_copy(p_v, out_hbm.at[pl.ds(wid * (ELG * B) + g * B, B)])
        return carry

    lax.fori_loop(0, ELG, chunk, 0)


_dot_sc = pl.kernel(
    _dot_body,
    out_type=jax.ShapeDtypeStruct((ELP, HW), jnp.float32),
    mesh=_mesh,
    scratch_types=[
        pltpu.VMEM((ELG, B), jnp.int32),
        pltpu.VMEM((ELG, B), jnp.int32),
        pltpu.VMEM((B, HW), jnp.float32),
        pltpu.VMEM((B, HW), jnp.float32),
        pltpu.VMEM((B, HW), jnp.float32),
        pltpu.VMEM((B, HW), jnp.float32),
        pltpu.VMEM((B, HW), jnp.float32),
        pltpu.SemaphoreType.DMA,
    ],
)


def _rowsum_body(p_ref, o_ref):
    o_ref[...] = jnp.sum(p_ref[...], axis=1, keepdims=True)


def _rowsum_tc(p):
    RB = 2048
    return pl.pallas_call(
        _rowsum_body,
        grid=(ELP // RB,),
        in_specs=[pl.BlockSpec((RB, HW), lambda i: (i, 0))],
        out_specs=pl.BlockSpec((RB, 1), lambda i: (i, 0)),
        out_shape=jax.ShapeDtypeStruct((ELP, 1), jnp.float32),
    )(p)


def _sage_body(relu, x_ref, s0_ref, s1_ref, cnt_ref, ws_ref, wn_ref, b_ref,
               o_ref):
    inv = 1.0 / jnp.maximum(cnt_ref[0, :, 0:1], 1.0)
    agg = jnp.concatenate([s0_ref[0], s1_ref[0]], axis=1) * inv
    y = (jnp.dot(x_ref[...], ws_ref[...], preferred_element_type=jnp.float32)
         + jnp.dot(agg, wn_ref[...], preferred_element_type=jnp.float32)
         + b_ref[...])
    o_ref[...] = jnp.maximum(y, 0.0) if relu else y


def _sage_tc(x, S, cnt, hsel, ws, wn, b, relu):
    RB = 1000
    return pl.pallas_call(
        functools.partial(_sage_body, relu),
        grid=(N // RB,),
        in_specs=[
            pl.BlockSpec((RB, H), lambda i: (i, 0)),
            pl.BlockSpec((1, RB, HW), lambda i: (0, i, 0)),
            pl.BlockSpec((1, RB, HW), lambda i: (1, i, 0)),
            pl.BlockSpec((1, RB, HW), lambda i, hsel=hsel: (hsel, i, 0)),
            pl.BlockSpec((H, H), lambda i: (0, 0)),
            pl.BlockSpec((H, H), lambda i: (0, 0)),
            pl.BlockSpec((1, H), lambda i: (0, 0)),
        ],
        out_specs=pl.BlockSpec((RB, H), lambda i: (i, 0)),
        out_shape=jax.ShapeDtypeStruct((N, H), jnp.float32),
    )(x, S, S, cnt, ws, wn, b.reshape(1, H))


def kernel(user_emb, movie_emb, W_self_um1, W_nei_um1, W_self_mu1, W_nei_mu1,
           W_self_um2, W_nei_um2, W_self_mu2, W_nei_mu2, b_m1, b_u1, b_m2,
           b_u2, edge_index, edge_label_index):
    src = edge_index[0].astype(jnp.int32)
    dst = edge_index[1].astype(jnp.int32)
    pad_g = jnp.zeros((EP - E,), jnp.int32)
    pad_s = jnp.full((EP - E,), N, jnp.int32)
    src_g = jnp.concatenate([src, pad_g]).reshape(NS, G, B)
    src_s = jnp.concatenate([src, pad_s]).reshape(NS, G, B)
    dst_g = jnp.concatenate([dst, pad_g]).reshape(NS, G, B)
    dst_s = jnp.concatenate([dst, pad_s]).reshape(NS, G, B)
    zeros = jnp.zeros((NP, HW), jnp.float32)
    ones = jnp.ones((B, HW), jnp.float32)

    u0, u1 = _bf_halves(user_emb)
    m0, m1 = _bf_halves(movie_emb)

    cnt = _cnt_sc(dst_s, src_s, zeros, ones)

    S_m = _agg_sc(u0, u1, src_g, dst_s, zeros)
    S_u = _agg_sc(m0, m1, dst_g, src_s, zeros)
    h_m = _sage_tc(movie_emb, S_m, cnt, 0, W_self_um1, W_nei_um1, b_m1, True)
    h_u = _sage_tc(user_emb, S_u, cnt, 1, W_self_mu1, W_nei_mu1, b_u1, True)

    hu0, hu1 = _bf_halves(h_u)
    hm0, hm1 = _bf_halves(h_m)
    S_m2 = _agg_sc(hu0, hu1, src_g, dst_s, zeros)
    S_u2 = _agg_sc(hm0, hm1, dst_g, src_s, zeros)
    h_m2 = _sage_tc(h_m, S_m2, cnt, 0, W_self_um2, W_nei_um2, b_m2, False)
    h_u2 = _sage_tc(h_u, S_u2, cnt, 1, W_self_mu2, W_nei_mu2, b_u2, False)

    eli_pad = jnp.zeros((ELP - EL,), jnp.int32)
    aidx = jnp.concatenate([edge_label_index[0].astype(jnp.int32), eli_pad])
    bidx = jnp.concatenate([edge_label_index[1].astype(jnp.int32), eli_pad])
    prod = _dot_sc(h_u2[:, :HW], h_u2[:, HW:], h_m2[:, :HW], h_m2[:, HW:],
                   aidx.reshape(NC * NS, ELG, B), bidx.reshape(NC * NS, ELG, B))
    pred = _rowsum_tc(prod)
    return pred[:EL, 0]
